# baseline (speedup 1.0000x reference)
import functools

import jax
import jax.numpy as jnp
from jax import lax
from jax.experimental import pallas as pl
from jax.experimental.pallas import tpu as pltpu
from jax.experimental.pallas import tpu_sc as plsc

B = 4096
L = 50
D = 64
DP = 128
BL = B * L
VOCAB = 1000000

_info = plsc.get_sparse_core_info()
NC, NS, LANES = _info.num_cores, _info.num_subcores, _info.num_lanes
NW = NC * NS
TOK_PER_W = BL // NW
CHUNK = 128
NCHUNK = TOK_PER_W // CHUNK
NBUF = 4
NQUAD = NCHUNK // NBUF


def _make_kernel():
    mesh = plsc.VectorSubcoreMesh(core_axis_name="c", subcore_axis_name="s")

    @functools.partial(
        pl.kernel,
        mesh=mesh,
        out_type=jax.ShapeDtypeStruct((BL, D), jnp.float32),
        compiler_params=pltpu.CompilerParams(use_tc_tiling_on_sc=False),
        scratch_types=[
            pltpu.VMEM((NCHUNK, CHUNK), jnp.int32),
            pltpu.VMEM((NCHUNK, CHUNK), jnp.float32),
            [pltpu.VMEM((CHUNK, D), jnp.float32)] * NBUF,
            [pltpu.VMEM((CHUNK, D), jnp.float32)] * NBUF,
            [pltpu.SemaphoreType.DMA] * NBUF,
            [pltpu.SemaphoreType.DMA] * NBUF,
        ],
    )
    def k(tok_hbm, mask_hbm, table_hbm, out_hbm,
          tok_v, mask_v, rows, cmp, gsems, osems):
        w = lax.axis_index("s") * NC + lax.axis_index("c")
        base_w = w * TOK_PER_W
        pltpu.sync_copy(tok_hbm.at[w], tok_v)
        pltpu.sync_copy(mask_hbm.at[w], mask_v)

        def gstart(c, rows, gsem):
            pltpu.async_copy(table_hbm.at[tok_v.at[c]], rows, gsem)

        def gwait(c, rows, gsem):
            pltpu.make_async_copy(table_hbm.at[tok_v.at[c]], rows, gsem).wait()

        def ostart(c, cmp, osem):
            pltpu.async_copy(cmp, out_hbm.at[pl.ds(base_w + c * CHUNK, CHUNK)],
                             osem)

        def owait(cmp, osem):
            pltpu.make_async_copy(cmp, out_hbm.at[pl.ds(base_w, CHUNK)],
                                  osem).wait()

        def compute(c, rows, cmp):
            for g in range(CHUNK // LANES):
                mk16 = mask_v[c, pl.ds(g * LANES, LANES)]
                for j in range(LANES):
                    t = g * LANES + j
                    m = lax.gather(
                        mk16,
                        jnp.full((LANES, 1), j, jnp.int32),
                        lax.GatherDimensionNumbers(
                            offset_dims=(), collapsed_slice_dims=(0,),
                            start_index_map=(0,)),
                        (1,),
                        mode=lax.GatherScatterMode.PROMISE_IN_BOUNDS)
                    for q in range(D // LANES):
                        sl = pl.ds(q * LANES, LANES)
                        cmp[t, sl] = rows[t, sl] * m

        for i in range(NBUF):
            gstart(i, rows[i], gsems[i])

        def slot(p, c, i):
            @pl.when(c < NCHUNK)
            def _():
                gwait(c, rows[i], gsems[i])

                @pl.when(p > 0)
                def _():
                    owait(cmp[i], osems[i])

                compute(c, rows[i], cmp[i])

                @pl.when(c + NBUF < NCHUNK)
                def _():
                    gstart(c + NBUF, rows[i], gsems[i])

                ostart(c, cmp[i], osems[i])

        def quad_body(p, carry):
            for i in range(NBUF):
                slot(p, NBUF * p + i, i)
            return carry

        lax.fori_loop(0, NQUAD + 1, quad_body, 0)
        for i in range(NBUF):
            owait(cmp[i], osems[i])

    return k


_sc_kernel = _make_kernel()


def kernel(token_ids, seq_lens, glove_table):
    table_p = jnp.pad(glove_table, ((0, 0), (0, DP - D)))
    table_v = table_p.reshape(2 * VOCAB, D)
    tok = (token_ids.reshape(NW, NCHUNK, CHUNK).astype(jnp.int32)) * 2
    mask = (jnp.arange(L, dtype=jnp.int32)[None, :]
            < seq_lens.astype(jnp.int32)[:, None]).astype(jnp.float32)
    mask3d = mask.reshape(NW, NCHUNK, CHUNK)
    out = _sc_kernel(tok, mask3d, table_v)
    return out.reshape(B, L, D)


# speedup vs baseline: 1.0096x; 1.0096x over previous
import functools

import jax
import jax.numpy as jnp
from jax import lax
from jax.experimental import pallas as pl
from jax.experimental.pallas import tpu as pltpu
from jax.experimental.pallas import tpu_sc as plsc

B = 4096
L = 50
D = 64
DP = 128
BL = B * L
VOCAB = 1000000

_info = plsc.get_sparse_core_info()
NC, NS, LANES = _info.num_cores, _info.num_subcores, _info.num_lanes
NW = NC * NS
TOK_PER_W = BL // NW
CHUNK = 128
NCHUNK = TOK_PER_W // CHUNK
NPAIR = NCHUNK // 2


def _make_kernel():
    mesh = plsc.VectorSubcoreMesh(core_axis_name="c", subcore_axis_name="s")

    @functools.partial(
        pl.kernel,
        mesh=mesh,
        out_type=jax.ShapeDtypeStruct((BL, D), jnp.float32),
        compiler_params=pltpu.CompilerParams(use_tc_tiling_on_sc=False),
        scratch_types=[
            pltpu.VMEM((NCHUNK, CHUNK), jnp.int32),
            pltpu.VMEM((NCHUNK, CHUNK), jnp.float32),
            pltpu.VMEM((CHUNK, D), jnp.float32),
            pltpu.VMEM((CHUNK, D), jnp.float32),
            pltpu.VMEM((CHUNK, D), jnp.float32),
            pltpu.VMEM((CHUNK, D), jnp.float32),
            pltpu.SemaphoreType.DMA,
            pltpu.SemaphoreType.DMA,
            pltpu.SemaphoreType.DMA,
            pltpu.SemaphoreType.DMA,
        ],
    )
    def k(tok_hbm, mask_hbm, table_hbm, out_hbm,
          tok_v, mask_v, rows_a, rows_b, cmp_a, cmp_b,
          gsem_a, gsem_b, osem_a, osem_b):
        w = lax.axis_index("s") * NC + lax.axis_index("c")
        base_w = w * TOK_PER_W
        pltpu.sync_copy(tok_hbm.at[w], tok_v)
        pltpu.sync_copy(mask_hbm.at[w], mask_v)

        def gstart(c, rows, gsem):
            pltpu.async_copy(table_hbm.at[tok_v.at[c]], rows, gsem)

        def gwait(c, rows, gsem):
            pltpu.make_async_copy(table_hbm.at[tok_v.at[c]], rows, gsem).wait()

        def ostart(c, cmp, osem):
            pltpu.async_copy(cmp, out_hbm.at[pl.ds(base_w + c * CHUNK, CHUNK)],
                             osem)

        def owait(cmp, osem):
            pltpu.make_async_copy(cmp, out_hbm.at[pl.ds(base_w, CHUNK)],
                                  osem).wait()

        def compute(c, rows, cmp):
            for g in range(CHUNK // LANES):
                mk16 = mask_v[c, pl.ds(g * LANES, LANES)]
                for j in range(LANES):
                    t = g * LANES + j
                    m = lax.gather(
                        mk16,
                        jnp.full((LANES, 1), j, jnp.int32),
                        lax.GatherDimensionNumbers(
                            offset_dims=(), collapsed_slice_dims=(0,),
                            start_index_map=(0,)),
                        (1,),
                        mode=lax.GatherScatterMode.PROMISE_IN_BOUNDS)
                    for q in range(D // LANES):
                        sl = pl.ds(q * LANES, LANES)
                        cmp[t, sl] = rows[t, sl] * m

        gstart(0, rows_a, gsem_a)
        gstart(1, rows_b, gsem_b)

        def half(p, c, rows, cmp, gsem, osem):
            gwait(c, rows, gsem)

            @pl.when(p > 0)
            def _():
                owait(cmp, osem)

            compute(c, rows, cmp)

            @pl.when(c + 2 < NCHUNK)
            def _():
                gstart(c + 2, rows, gsem)

            ostart(c, cmp, osem)

        def pair_body(p, carry):
            half(p, 2 * p, rows_a, cmp_a, gsem_a, osem_a)
            half(p, 2 * p + 1, rows_b, cmp_b, gsem_b, osem_b)
            return carry

        lax.fori_loop(0, NPAIR, pair_body, 0)
        owait(cmp_a, osem_a)
        owait(cmp_b, osem_b)

    return k


_sc_kernel = _make_kernel()


def kernel(token_ids, seq_lens, glove_table):
    table_p = jnp.pad(glove_table, ((0, 0), (0, DP - D)))
    table_v = table_p.reshape(2 * VOCAB, D)
    tok = (token_ids.reshape(NW, NCHUNK, CHUNK).astype(jnp.int32)) * 2
    mask = (jnp.arange(L, dtype=jnp.int32)[None, :]
            < seq_lens.astype(jnp.int32)[:, None]).astype(jnp.float32)
    mask3d = mask.reshape(NW, NCHUNK, CHUNK)
    out = _sc_kernel(tok, mask3d, table_v)
    return out.reshape(B, L, D)
